# baseline (speedup 1.0000x reference)
import functools

import jax
import jax.numpy as jnp
from jax import lax
from jax.experimental import pallas as pl
from jax.experimental.pallas import tpu as pltpu
from jax.experimental.pallas import tpu_sc as plsc

_B = 4096
_F = 196
_D = 16
_NW = 32
_RPW = _B // _NW
_WSTRIDE = _RPW * _F
_LROW = _WSTRIDE // 128
_PAIR = 2 * _F
_NPAIR = _RPW // 2
_CHUNKS = ((0, 128), (128, 128), (256, 128), (384, 8))
_NACC = 4
_NG = 13


def _issue_pair(table_hbm, idx_f, rows_v, buf, p, sem):
  o = p * _PAIR
  for co, cs in _CHUNKS:
    pltpu.async_copy(
        table_hbm.at[idx_f.at[pl.ds(o + co, cs)]],
        rows_v.at[buf, pl.ds(co, cs)], sem)


def _wait_pair(table_hbm, idx_f, rows_v, buf, sem):
  for co, cs in _CHUNKS:
    pltpu.make_async_copy(
        table_hbm.at[idx_f.at[pl.ds(co, cs)]],
        rows_v.at[buf, pl.ds(co, cs)], sem).wait()


def _compute_pair(vals_f, rows_v, wfm_regs, buf, p, s_v, q_v, f_v):
  for half in range(2):
    r = 2 * p + half
    base = half * _F
    voff = r * _F
    sa = [jnp.zeros((_D,), jnp.float32) for _ in range(_NACC)]
    qa = [jnp.zeros((_D,), jnp.float32) for _ in range(_NACC)]
    fm0 = jnp.zeros((_D,), jnp.float32)
    fm1 = jnp.zeros((_D,), jnp.float32)
    for g in range(_NG):
      v16 = vals_f[pl.ds(voff + 16 * g, _D)]
      if g % 2 == 0:
        fm0 = fm0 + v16 * wfm_regs[g]
      else:
        fm1 = fm1 + v16 * wfm_regs[g]
      for j in range(min(16, _F - 16 * g)):
        f = 16 * g + j
        t = v16[j] * rows_v[buf, base + f, :]
        sa[f % _NACC] = sa[f % _NACC] + t
        qa[f % _NACC] = qa[f % _NACC] + t * t
    s_v[r, :] = (sa[0] + sa[1]) + (sa[2] + sa[3])
    q_v[r, :] = (qa[0] + qa[1]) + (qa[2] + qa[3])
    f_v[r, :] = fm0 + fm1


def _pool_body(ids_hbm, vals_hbm, table_hbm, wfm_hbm, s_hbm, q_hbm, fm_hbm,
               idx_f, vals_f, wfm_v, rows_v, s_v, q_v, f_v, sem0, sem1):
  wid = lax.axis_index("s") * 2 + lax.axis_index("c")
  wrow = wid * _LROW

  def stage(k, carry):
    pltpu.async_copy(ids_hbm.at[wrow + k, :],
                     idx_f.at[pl.ds(128 * k, 128)], sem0)
    pltpu.async_copy(vals_hbm.at[wrow + k, :],
                     vals_f.at[pl.ds(128 * k, 128)], sem0)
    return carry

  lax.fori_loop(0, _LROW, stage, 0)

  def drain(k, carry):
    pltpu.make_async_copy(ids_hbm.at[0, :],
                          idx_f.at[pl.ds(0, 128)], sem0).wait()
    pltpu.make_async_copy(vals_hbm.at[0, :],
                          vals_f.at[pl.ds(0, 128)], sem0).wait()
    return carry

  lax.fori_loop(0, _LROW, drain, 0)

  vals_f[pl.ds(_WSTRIDE, _D)] = jnp.zeros((_D,), jnp.float32)
  pltpu.sync_copy(wfm_hbm, wfm_v)

  wfm_regs = [wfm_v[g // 8, pl.ds(16 * (g % 8), _D)] for g in range(_NG)]

  _issue_pair(table_hbm, idx_f, rows_v, 0, 0, sem0)

  def body(i, carry):
    p0 = 2 * i
    _issue_pair(table_hbm, idx_f, rows_v, 1, p0 + 1, sem1)
    _wait_pair(table_hbm, idx_f, rows_v, 0, sem0)
    _compute_pair(vals_f, rows_v, wfm_regs, 0, p0, s_v, q_v, f_v)

    @pl.when(i < _NPAIR // 2 - 1)
    def _():
      _issue_pair(table_hbm, idx_f, rows_v, 0, p0 + 2, sem0)

    _wait_pair(table_hbm, idx_f, rows_v, 1, sem1)
    _compute_pair(vals_f, rows_v, wfm_regs, 1, p0 + 1, s_v, q_v, f_v)
    return carry

  lax.fori_loop(0, _NPAIR // 2, body, 0)

  pltpu.sync_copy(s_v, s_hbm.at[pl.ds(wid * _RPW, _RPW)])
  pltpu.sync_copy(q_v, q_hbm.at[pl.ds(wid * _RPW, _RPW)])
  pltpu.sync_copy(f_v, fm_hbm.at[pl.ds(wid * _RPW, _RPW)])


@functools.cache
def _make_pool():
  return functools.partial(
      pl.kernel,
      out_type=(jax.ShapeDtypeStruct((_B, _D), jnp.float32),
                jax.ShapeDtypeStruct((_B, _D), jnp.float32),
                jax.ShapeDtypeStruct((_B, _D), jnp.float32)),
      mesh=plsc.VectorSubcoreMesh(core_axis_name="c", subcore_axis_name="s"),
      scratch_types=[
          pltpu.VMEM((_WSTRIDE,), jnp.int32),
          pltpu.VMEM((_WSTRIDE + _D,), jnp.float32),
          pltpu.VMEM((2, 128), jnp.float32),
          pltpu.VMEM((2, _PAIR, _D), jnp.float32),
          pltpu.VMEM((_RPW, _D), jnp.float32),
          pltpu.VMEM((_RPW, _D), jnp.float32),
          pltpu.VMEM((_RPW, _D), jnp.float32),
          pltpu.SemaphoreType.DMA,
          pltpu.SemaphoreType.DMA,
      ],
      compiler_params=pltpu.CompilerParams(
          needs_layout_passes=False, use_tc_tiling_on_sc=False),
  )(_pool_body)


_V = 1000000
_TCOLS = 8192


def _tr_body(t_ref, o_ref):
  o_ref[...] = t_ref[...].T


@functools.cache
def _make_transpose():
  grid = (_V + _TCOLS - 1) // _TCOLS
  return pl.pallas_call(
      _tr_body,
      grid=(grid,),
      in_specs=[pl.BlockSpec((_D, _TCOLS), lambda i: (0, i))],
      out_specs=pl.BlockSpec((_TCOLS, _D), lambda i: (i, 0)),
      out_shape=jax.ShapeDtypeStruct((_V, _D), jnp.float32),
  )


def _mlp_body(s_ref, q_ref, fmp_ref, w1_ref, b1_ref, g1_ref, be1_ref,
              m1_ref, v1_ref, w2_ref, b2_ref, g2_ref, be2_ref, m2_ref,
              v2_ref, wd_ref, bfm_ref, o_ref):
  s = s_ref[...]
  q = q_ref[...]
  bi = 0.5 * (s * s) - q
  h = jnp.dot(bi, w1_ref[...], preferred_element_type=jnp.float32)
  h = jnp.maximum(h + b1_ref[...], 0.0)
  h = (h - m1_ref[...]) * (g1_ref[...] * lax.rsqrt(v1_ref[...] + 1e-3))
  h = h + be1_ref[...]
  h = jnp.dot(h, w2_ref[...], preferred_element_type=jnp.float32)
  h = jnp.maximum(h + b2_ref[...], 0.0)
  h = (h - m2_ref[...]) * (g2_ref[...] * lax.rsqrt(v2_ref[...] + 1e-3))
  h = h + be2_ref[...]
  x = jnp.sum(h * wd_ref[...], axis=1, keepdims=True)
  fm = jnp.sum(fmp_ref[...], axis=1, keepdims=True)
  o_ref[...] = jax.nn.sigmoid(x + fm + bfm_ref[...])


def kernel(feature_ids, feature_vals, table, W1, b1, g1, be1, m1, v1,
           W2, b2, g2, be2, m2, v2, Wd, Wfm, bfm):
  ids2 = feature_ids.reshape(_B * _F // 128, 128)
  vals2 = feature_vals.reshape(_B * _F // 128, 128)
  wfm2 = jnp.pad(Wfm.reshape(-1), (0, 256 - _F)).reshape(2, 128)

  table_lin = _make_transpose()(table.T)
  s_mom, q_mom, fm_part = _make_pool()(ids2, vals2, table_lin, wfm2)

  out = pl.pallas_call(
      _mlp_body,
      out_shape=jax.ShapeDtypeStruct((_B, 1), jnp.float32),
  )(s_mom, q_mom, fm_part,
    W1, b1.reshape(1, -1), g1.reshape(1, -1), be1.reshape(1, -1),
    m1.reshape(1, -1), v1.reshape(1, -1),
    W2, b2.reshape(1, -1), g2.reshape(1, -1), be2.reshape(1, -1),
    m2.reshape(1, -1), v2.reshape(1, -1),
    Wd.reshape(1, -1), bfm.reshape(1, 1))
  return out


# speedup vs baseline: 1.8906x; 1.8906x over previous
import functools

import jax
import jax.numpy as jnp
from jax import lax
from jax.experimental import pallas as pl
from jax.experimental.pallas import tpu as pltpu
from jax.experimental.pallas import tpu_sc as plsc

_B = 4096
_F = 196
_D = 16
_NW = 32
_RPW = _B // _NW
_WSTRIDE = _RPW * _F
_LROW = _WSTRIDE // 128
_PAIR = 2 * _F
_NPAIR = _RPW // 2
_CHUNKS = ((0, 128), (128, 128), (256, 128), (384, 8))
_NACC = 4
_NG = 13


def _issue_pair(table_hbm, idx_f, rows_v, buf, p, sem):
  o = p * _PAIR
  for co, cs in _CHUNKS:
    pltpu.async_copy(
        table_hbm.at[idx_f.at[pl.ds(o + co, cs)]],
        rows_v.at[buf, pl.ds(co, cs)], sem)


def _wait_pair(table_hbm, idx_f, rows_v, buf, sem):
  for co, cs in _CHUNKS:
    pltpu.make_async_copy(
        table_hbm.at[idx_f.at[pl.ds(co, cs)]],
        rows_v.at[buf, pl.ds(co, cs)], sem).wait()


def _compute_pair(vals_f, rows_v, wfm_regs, buf, p, s_v, q_v, f_v):
  for half in range(2):
    r = 2 * p + half
    base = half * _F
    voff = r * _F
    sa = [jnp.zeros((_D,), jnp.float32) for _ in range(_NACC)]
    qa = [jnp.zeros((_D,), jnp.float32) for _ in range(_NACC)]
    fm0 = jnp.zeros((_D,), jnp.float32)
    fm1 = jnp.zeros((_D,), jnp.float32)
    for g in range(_NG):
      v16 = vals_f[pl.ds(voff + 16 * g, _D)]
      if g % 2 == 0:
        fm0 = fm0 + v16 * wfm_regs[g]
      else:
        fm1 = fm1 + v16 * wfm_regs[g]
      for j in range(min(16, _F - 16 * g)):
        f = 16 * g + j
        t = v16[j] * rows_v[buf, base + f, :]
        sa[f % _NACC] = sa[f % _NACC] + t
        qa[f % _NACC] = qa[f % _NACC] + t * t
    s_v[r, :] = (sa[0] + sa[1]) + (sa[2] + sa[3])
    q_v[r, :] = (qa[0] + qa[1]) + (qa[2] + qa[3])
    f_v[r, :] = fm0 + fm1


def _pool_body(ids_hbm, vals_hbm, table_hbm, wfm_hbm, s_hbm, q_hbm, fm_hbm,
               idx_f, vals_f, wfm_v, rows_v, s_v, q_v, f_v, sem0, sem1):
  wid = lax.axis_index("s") * 2 + lax.axis_index("c")
  wrow = wid * _LROW

  def stage(k, carry):
    pltpu.async_copy(ids_hbm.at[wrow + k, :],
                     idx_f.at[pl.ds(128 * k, 128)], sem0)
    pltpu.async_copy(vals_hbm.at[wrow + k, :],
                     vals_f.at[pl.ds(128 * k, 128)], sem0)
    return carry

  lax.fori_loop(0, _LROW, stage, 0)

  def drain(k, carry):
    pltpu.make_async_copy(ids_hbm.at[0, :],
                          idx_f.at[pl.ds(0, 128)], sem0).wait()
    pltpu.make_async_copy(vals_hbm.at[0, :],
                          vals_f.at[pl.ds(0, 128)], sem0).wait()
    return carry

  lax.fori_loop(0, _LROW, drain, 0)

  vals_f[pl.ds(_WSTRIDE, _D)] = jnp.zeros((_D,), jnp.float32)
  pltpu.sync_copy(wfm_hbm, wfm_v)

  wfm_regs = [wfm_v[g // 8, pl.ds(16 * (g % 8), _D)] for g in range(_NG)]

  _issue_pair(table_hbm, idx_f, rows_v, 0, 0, sem0)

  def body(i, carry):
    p0 = 2 * i
    _issue_pair(table_hbm, idx_f, rows_v, 1, p0 + 1, sem1)
    _wait_pair(table_hbm, idx_f, rows_v, 0, sem0)
    _compute_pair(vals_f, rows_v, wfm_regs, 0, p0, s_v, q_v, f_v)

    @pl.when(i < _NPAIR // 2 - 1)
    def _():
      _issue_pair(table_hbm, idx_f, rows_v, 0, p0 + 2, sem0)

    _wait_pair(table_hbm, idx_f, rows_v, 1, sem1)
    _compute_pair(vals_f, rows_v, wfm_regs, 1, p0 + 1, s_v, q_v, f_v)
    return carry

  lax.fori_loop(0, _NPAIR // 2, body, 0)

  pltpu.sync_copy(s_v, s_hbm.at[pl.ds(wid * _RPW, _RPW)])
  pltpu.sync_copy(q_v, q_hbm.at[pl.ds(wid * _RPW, _RPW)])
  pltpu.sync_copy(f_v, fm_hbm.at[pl.ds(wid * _RPW, _RPW)])


@functools.cache
def _make_pool():
  return functools.partial(
      pl.kernel,
      out_type=(jax.ShapeDtypeStruct((_B, _D), jnp.float32),
                jax.ShapeDtypeStruct((_B, _D), jnp.float32),
                jax.ShapeDtypeStruct((_B, _D), jnp.float32)),
      mesh=plsc.VectorSubcoreMesh(core_axis_name="c", subcore_axis_name="s"),
      scratch_types=[
          pltpu.VMEM((_WSTRIDE,), jnp.int32),
          pltpu.VMEM((_WSTRIDE + _D,), jnp.float32),
          pltpu.VMEM((2, 128), jnp.float32),
          pltpu.VMEM((2, _PAIR, _D), jnp.float32),
          pltpu.VMEM((_RPW, _D), jnp.float32),
          pltpu.VMEM((_RPW, _D), jnp.float32),
          pltpu.VMEM((_RPW, _D), jnp.float32),
          pltpu.SemaphoreType.DMA,
          pltpu.SemaphoreType.DMA,
      ],
      compiler_params=pltpu.CompilerParams(
          needs_layout_passes=False, use_tc_tiling_on_sc=False),
  )(_pool_body)


_V = 1000000
_TCH = 8192
_TBLK = 8 * _TCH
_TGRID = (_V + _TBLK - 1) // _TBLK


def _tr_body(t_ref, o_ref):
  o_ref[:, 0:_D] = t_ref[...].T


@functools.cache
def _make_transpose():
  grid = (_V + _TCH - 1) // _TCH
  return pl.pallas_call(
      _tr_body,
      grid=(grid,),
      in_specs=[pl.BlockSpec((_D, _TCH), lambda i: (0, i))],
      out_specs=pl.BlockSpec((_TCH, 128), lambda i: (i, 0)),
      out_shape=jax.ShapeDtypeStruct((((_V + _TCH - 1) // _TCH) * _TCH, 128),
                                     jnp.float32),
  )


def _mlp_body(s_ref, q_ref, fmp_ref, w1_ref, b1_ref, g1_ref, be1_ref,
              m1_ref, v1_ref, w2_ref, b2_ref, g2_ref, be2_ref, m2_ref,
              v2_ref, wd_ref, bfm_ref, o_ref):
  s = s_ref[...]
  q = q_ref[...]
  bi = 0.5 * (s * s) - q
  h = jnp.dot(bi, w1_ref[...], preferred_element_type=jnp.float32)
  h = jnp.maximum(h + b1_ref[...], 0.0)
  h = (h - m1_ref[...]) * (g1_ref[...] * lax.rsqrt(v1_ref[...] + 1e-3))
  h = h + be1_ref[...]
  h = jnp.dot(h, w2_ref[...], preferred_element_type=jnp.float32)
  h = jnp.maximum(h + b2_ref[...], 0.0)
  h = (h - m2_ref[...]) * (g2_ref[...] * lax.rsqrt(v2_ref[...] + 1e-3))
  h = h + be2_ref[...]
  x = jnp.sum(h * wd_ref[...], axis=1, keepdims=True)
  fm = jnp.sum(fmp_ref[...], axis=1, keepdims=True)
  o_ref[...] = jax.nn.sigmoid(x + fm + bfm_ref[...])


def kernel(feature_ids, feature_vals, table, W1, b1, g1, be1, m1, v1,
           W2, b2, g2, be2, m2, v2, Wd, Wfm, bfm):
  ids2 = feature_ids.reshape(_B * _F // 128, 128) * 8
  vals2 = feature_vals.reshape(_B * _F // 128, 128)
  wfm2 = jnp.pad(Wfm.reshape(-1), (0, 256 - _F)).reshape(2, 128)

  tpad = _make_transpose()(table.T)
  table_lin = tpad.reshape(tpad.shape[0] * 8, _D)
  s_mom, q_mom, fm_part = _make_pool()(ids2, vals2, table_lin, wfm2)

  out = pl.pallas_call(
      _mlp_body,
      out_shape=jax.ShapeDtypeStruct((_B, 1), jnp.float32),
  )(s_mom, q_mom, fm_part,
    W1, b1.reshape(1, -1), g1.reshape(1, -1), be1.reshape(1, -1),
    m1.reshape(1, -1), v1.reshape(1, -1),
    W2, b2.reshape(1, -1), g2.reshape(1, -1), be2.reshape(1, -1),
    m2.reshape(1, -1), v2.reshape(1, -1),
    Wd.reshape(1, -1), bfm.reshape(1, 1))
  return out


# speedup vs baseline: 3.7393x; 1.9778x over previous
import functools

import jax
import jax.numpy as jnp
from jax import lax
from jax.experimental import pallas as pl
from jax.experimental.pallas import tpu as pltpu
from jax.experimental.pallas import tpu_sc as plsc

_B = 4096
_F = 196
_D = 16
_NW = 32
_RPW = _B // _NW
_WSTRIDE = _RPW * _F
_LROW = _WSTRIDE // 128
_PAIR = 2 * _F
_NPAIR = _RPW // 2
_CHUNKS = ((0, 128), (128, 128), (256, 128), (384, 8))
_NACC = 4
_NG = 13


def _issue_pair(table_hbm, idx_f, rows_v, buf, p, sem):
  o = p * _PAIR
  for co, cs in _CHUNKS:
    pltpu.async_copy(
        table_hbm.at[idx_f.at[pl.ds(o + co, cs)]],
        rows_v.at[buf, pl.ds(co, cs)], sem)


def _wait_pair(table_hbm, idx_f, rows_v, buf, sem):
  for co, cs in _CHUNKS:
    pltpu.make_async_copy(
        table_hbm.at[idx_f.at[pl.ds(co, cs)]],
        rows_v.at[buf, pl.ds(co, cs)], sem).wait()


def _compute_pair(vals_f, rows_v, wfm_regs, buf, p, s_v, q_v, f_v):
  for half in range(2):
    r = 2 * p + half
    base = half * _F
    voff = r * _F
    sa = [jnp.zeros((_D,), jnp.float32) for _ in range(_NACC)]
    qa = [jnp.zeros((_D,), jnp.float32) for _ in range(_NACC)]
    fm0 = jnp.zeros((_D,), jnp.float32)
    fm1 = jnp.zeros((_D,), jnp.float32)
    for g in range(_NG):
      v16 = vals_f[pl.ds(voff + 16 * g, _D)]
      if g % 2 == 0:
        fm0 = fm0 + v16 * wfm_regs[g]
      else:
        fm1 = fm1 + v16 * wfm_regs[g]
      for j in range(min(16, _F - 16 * g)):
        f = 16 * g + j
        t = v16[j] * rows_v[buf, base + f, :]
        sa[f % _NACC] = sa[f % _NACC] + t
        qa[f % _NACC] = qa[f % _NACC] + t * t
    s_v[r, :] = (sa[0] + sa[1]) + (sa[2] + sa[3])
    q_v[r, :] = (qa[0] + qa[1]) + (qa[2] + qa[3])
    f_v[r, :] = fm0 + fm1


def _pool_body(ids_hbm, vals_hbm, table_hbm, wfm_hbm, s_hbm, q_hbm, fm_hbm,
               idx_f, vals_f, wfm_v, rows_v, s_v, q_v, f_v, sem0, sem1):
  wid = lax.axis_index("s") * 2 + lax.axis_index("c")
  wrow = wid * _LROW

  def stage(k, carry):
    pltpu.async_copy(ids_hbm.at[wrow + k, :],
                     idx_f.at[pl.ds(128 * k, 128)], sem0)
    pltpu.async_copy(vals_hbm.at[wrow + k, :],
                     vals_f.at[pl.ds(128 * k, 128)], sem0)
    return carry

  lax.fori_loop(0, _LROW, stage, 0)

  def drain(k, carry):
    pltpu.make_async_copy(ids_hbm.at[0, :],
                          idx_f.at[pl.ds(0, 128)], sem0).wait()
    pltpu.make_async_copy(vals_hbm.at[0, :],
                          vals_f.at[pl.ds(0, 128)], sem0).wait()
    return carry

  lax.fori_loop(0, _LROW, drain, 0)

  vals_f[pl.ds(_WSTRIDE, _D)] = jnp.zeros((_D,), jnp.float32)
  pltpu.sync_copy(wfm_hbm, wfm_v)

  wfm_regs = [wfm_v[g // 8, pl.ds(16 * (g % 8), _D)] for g in range(_NG)]

  _issue_pair(table_hbm, idx_f, rows_v, 0, 0, sem0)

  def body(i, carry):
    p0 = 2 * i
    _issue_pair(table_hbm, idx_f, rows_v, 1, p0 + 1, sem1)
    _wait_pair(table_hbm, idx_f, rows_v, 0, sem0)
    _compute_pair(vals_f, rows_v, wfm_regs, 0, p0, s_v, q_v, f_v)

    @pl.when(i < _NPAIR // 2 - 1)
    def _():
      _issue_pair(table_hbm, idx_f, rows_v, 0, p0 + 2, sem0)

    _wait_pair(table_hbm, idx_f, rows_v, 1, sem1)
    _compute_pair(vals_f, rows_v, wfm_regs, 1, p0 + 1, s_v, q_v, f_v)
    return carry

  lax.fori_loop(0, _NPAIR // 2, body, 0)

  pltpu.sync_copy(s_v, s_hbm.at[pl.ds(wid * _RPW, _RPW)])
  pltpu.sync_copy(q_v, q_hbm.at[pl.ds(wid * _RPW, _RPW)])
  pltpu.sync_copy(f_v, fm_hbm.at[pl.ds(wid * _RPW, _RPW)])


@functools.cache
def _make_pool():
  return functools.partial(
      pl.kernel,
      out_type=(jax.ShapeDtypeStruct((_B, _D), jnp.float32),
                jax.ShapeDtypeStruct((_B, _D), jnp.float32),
                jax.ShapeDtypeStruct((_B, _D), jnp.float32)),
      mesh=plsc.VectorSubcoreMesh(core_axis_name="c", subcore_axis_name="s"),
      scratch_types=[
          pltpu.VMEM((_WSTRIDE,), jnp.int32),
          pltpu.VMEM((_WSTRIDE + _D,), jnp.float32),
          pltpu.VMEM((2, 128), jnp.float32),
          pltpu.VMEM((2, _PAIR, _D), jnp.float32),
          pltpu.VMEM((_RPW, _D), jnp.float32),
          pltpu.VMEM((_RPW, _D), jnp.float32),
          pltpu.VMEM((_RPW, _D), jnp.float32),
          pltpu.SemaphoreType.DMA,
          pltpu.SemaphoreType.DMA,
      ],
      compiler_params=pltpu.CompilerParams(
          needs_layout_passes=False, use_tc_tiling_on_sc=False),
  )(_pool_body)


_V = 1000000
_TCH = 4096
_TBLK = 8 * _TCH
_TGRID = (_V + _TBLK - 1) // _TBLK


def _tr_body(t_ref, o_ref, x_ref):
  for a in range(8):
    x_ref[_D * a:_D * (a + 1), :] = t_ref[:, _TCH * a:_TCH * (a + 1)]
  o_ref[...] = x_ref[...].T


@functools.cache
def _make_transpose():
  return pl.pallas_call(
      _tr_body,
      grid=(_TGRID,),
      in_specs=[pl.BlockSpec((_D, _TBLK), lambda i: (0, i))],
      out_specs=pl.BlockSpec((_TCH, 128), lambda i: (i, 0)),
      out_shape=jax.ShapeDtypeStruct((_TGRID * _TCH, 128), jnp.float32),
      scratch_shapes=[pltpu.VMEM((128, _TCH), jnp.float32)],
  )


def _mlp_body(s_ref, q_ref, fmp_ref, w1_ref, b1_ref, g1_ref, be1_ref,
              m1_ref, v1_ref, w2_ref, b2_ref, g2_ref, be2_ref, m2_ref,
              v2_ref, wd_ref, bfm_ref, o_ref):
  s = s_ref[...]
  q = q_ref[...]
  bi = 0.5 * (s * s) - q
  h = jnp.dot(bi, w1_ref[...], preferred_element_type=jnp.float32)
  h = jnp.maximum(h + b1_ref[...], 0.0)
  h = (h - m1_ref[...]) * (g1_ref[...] * lax.rsqrt(v1_ref[...] + 1e-3))
  h = h + be1_ref[...]
  h = jnp.dot(h, w2_ref[...], preferred_element_type=jnp.float32)
  h = jnp.maximum(h + b2_ref[...], 0.0)
  h = (h - m2_ref[...]) * (g2_ref[...] * lax.rsqrt(v2_ref[...] + 1e-3))
  h = h + be2_ref[...]
  x = jnp.sum(h * wd_ref[...], axis=1, keepdims=True)
  fm = jnp.sum(fmp_ref[...], axis=1, keepdims=True)
  o_ref[...] = jax.nn.sigmoid(x + fm + bfm_ref[...])


def kernel(feature_ids, feature_vals, table, W1, b1, g1, be1, m1, v1,
           W2, b2, g2, be2, m2, v2, Wd, Wfm, bfm):
  v = feature_ids.reshape(_B * _F // 128, 128)
  ids2 = (((v // _TBLK) * _TCH + (v & (_TCH - 1))) << 3) + ((v // _TCH) & 7)
  vals2 = feature_vals.reshape(_B * _F // 128, 128)
  wfm2 = jnp.pad(Wfm.reshape(-1), (0, 256 - _F)).reshape(2, 128)

  tpad = _make_transpose()(table.T)
  table_lin = tpad.reshape(tpad.shape[0] * 8, _D)
  s_mom, q_mom, fm_part = _make_pool()(ids2, vals2, table_lin, wfm2)

  out = pl.pallas_call(
      _mlp_body,
      out_shape=jax.ShapeDtypeStruct((_B, 1), jnp.float32),
  )(s_mom, q_mom, fm_part,
    W1, b1.reshape(1, -1), g1.reshape(1, -1), be1.reshape(1, -1),
    m1.reshape(1, -1), v1.reshape(1, -1),
    W2, b2.reshape(1, -1), g2.reshape(1, -1), be2.reshape(1, -1),
    m2.reshape(1, -1), v2.reshape(1, -1),
    Wd.reshape(1, -1), bfm.reshape(1, 1))
  return out
